# baseline (speedup 1.0000x reference)
import jax
import jax.numpy as jnp
from jax import lax
from jax.experimental import pallas as pl
from jax.experimental.pallas import tpu as pltpu
from jax.experimental.pallas import tpu_sc as plsc

N_ENT = 10000
N_REL = 10000
IN_DIM = 256
OUT_DIM = 256
N_EDGES = 160000

NPAD = 10240
NC, NS, L = 2, 16, 16
CH = 32
EPT = 10240
EP = EPT * NS
DH = 128
BLK = 512
EBR = NPAD // DH
NQ = 10016
ACCR = 10016


def _mm_body(ent_ref, rel_ref, w1_ref, w2_ref, w3_ref, ab_ref, a2_ref,
             qb_ref, p1_ref, p2_ref, p3_ref, q1_ref, q2_ref, q3_ref):
    ent = ent_ref[...]
    rel = rel_ref[...]
    dn = (((1,), (1,)), ((), ()))
    p1 = lax.dot_general(ent, w1_ref[...], dn,
                         preferred_element_type=jnp.float32) + ab_ref[...]
    p2 = lax.dot_general(rel, w2_ref[...], dn,
                         preferred_element_type=jnp.float32)
    p3 = lax.dot_general(ent, w3_ref[...], dn,
                         preferred_element_type=jnp.float32)
    p1_ref[...] = p1
    p2_ref[...] = p2
    p3_ref[...] = p3
    a2 = a2_ref[...]
    q1 = jnp.sum(p1 * a2, axis=1, keepdims=True) + qb_ref[...]
    q2 = jnp.sum(p2 * a2, axis=1, keepdims=True)
    q3 = jnp.sum(p3 * a2, axis=1, keepdims=True)
    q1_ref[...] = jnp.broadcast_to(q1, (BLK, L))
    q2_ref[...] = jnp.broadcast_to(q2, (BLK, L))
    q3_ref[...] = jnp.broadcast_to(q3, (BLK, L))


def _node_matmuls(ent_p, rel_p, w1, w2, w3, ab2d, a2row, qb2d):
    nblk = NPAD // BLK
    return pl.pallas_call(
        _mm_body,
        grid=(nblk,),
        in_specs=[
            pl.BlockSpec((BLK, IN_DIM), lambda i: (i, 0)),
            pl.BlockSpec((BLK, IN_DIM), lambda i: (i, 0)),
            pl.BlockSpec((OUT_DIM, IN_DIM), lambda i: (0, 0)),
            pl.BlockSpec((OUT_DIM, IN_DIM), lambda i: (0, 0)),
            pl.BlockSpec((OUT_DIM, IN_DIM), lambda i: (0, 0)),
            pl.BlockSpec((1, OUT_DIM), lambda i: (0, 0)),
            pl.BlockSpec((1, OUT_DIM), lambda i: (0, 0)),
            pl.BlockSpec((1, 1), lambda i: (0, 0)),
        ],
        out_specs=[
            pl.BlockSpec((BLK, OUT_DIM), lambda i: (i, 0)),
            pl.BlockSpec((BLK, OUT_DIM), lambda i: (i, 0)),
            pl.BlockSpec((BLK, OUT_DIM), lambda i: (i, 0)),
            pl.BlockSpec((BLK, L), lambda i: (i, 0)),
            pl.BlockSpec((BLK, L), lambda i: (i, 0)),
            pl.BlockSpec((BLK, L), lambda i: (i, 0)),
        ],
        out_shape=[
            jax.ShapeDtypeStruct((NPAD, OUT_DIM), jnp.float32),
            jax.ShapeDtypeStruct((NPAD, OUT_DIM), jnp.float32),
            jax.ShapeDtypeStruct((NPAD, OUT_DIM), jnp.float32),
            jax.ShapeDtypeStruct((NPAD, L), jnp.float32),
            jax.ShapeDtypeStruct((NPAD, L), jnp.float32),
            jax.ShapeDtypeStruct((NPAD, L), jnp.float32),
        ],
    )(ent_p, rel_p, w1, w2, w3, ab2d, a2row, qb2d)


def _sc_body(idx3_hbm, q1_hbm, q2_hbm, q3_hbm, ptab_hbm, out_hbm, eb_hbm,
             q1v, q2v, q3v, icat, hbA, hbB, eiA, eiB, wbufA, wbufB,
             gidxA, gidxB, rowsA, rowsB, acc, acc_eb,
             semgA, semgB, semaA, semaB, sembA, sembB):
    c = lax.axis_index("c")
    s = lax.axis_index("s")
    iota = lax.iota(jnp.int32, L)
    zeros_i = jnp.zeros((L,), jnp.int32)
    zeros_f = jnp.zeros((L,), jnp.float32)
    p3base = 2 * NPAD

    pltpu.sync_copy(q1_hbm, q1v)
    pltpu.sync_copy(q2_hbm, q2v)
    pltpu.sync_copy(q3_hbm, q3v)

    def _zero_row(e, _):
        for j in range(DH // L):
            rowsA[e, pl.ds(j * L, L)] = zeros_f
            rowsB[e, pl.ds(j * L, L)] = zeros_f
        return 0
    lax.fori_loop(0, 2 * CH, _zero_row, 0)

    def _zero_acc(k, _):
        @pl.when(lax.rem(k, NS) == s)
        def _():
            pltpu.sync_copy(rowsA, acc.at[pl.ds(k * 2 * CH, 2 * CH)])
        return 0
    lax.fori_loop(0, ACCR // (2 * CH), _zero_acc, 0)

    @pl.when(s == 1)
    def _():
        rem = ACCR - (ACCR // (2 * CH)) * 2 * CH
        pltpu.sync_copy(rowsA.at[pl.ds(0, rem)],
                        acc.at[pl.ds(ACCR - rem, rem)])

    @pl.when(s == 0)
    def _():
        pltpu.sync_copy(rowsA, acc_eb.at[pl.ds(0, 2 * CH)])
        pltpu.sync_copy(rowsA.at[pl.ds(0, EBR - 2 * CH)],
                        acc_eb.at[pl.ds(2 * CH, EBR - 2 * CH)])
    for g in range(CH // L):
        sl = pl.ds(g * L, L)
        hbB[sl] = zeros_i
        eiB[sl] = zeros_i
    plsc.subcore_barrier()

    def _stage1(k, off, wbuf, gidx):
        if off == 0:
            pltpu.sync_copy(idx3_hbm.at[s].at[k // 2], icat)
        for g in range(CH // L):
            sl = pl.ds(off + g * L, L)
            hv = icat[0, sl]
            rv = icat[1, sl]
            tv = icat[2, sl]
            sc = (plsc.load_gather(q1v, [hv]) +
                  plsc.load_gather(q2v, [rv]) +
                  plsc.load_gather(q3v, [tv]))
            w = jnp.exp(jnp.maximum(sc, 0.01 * sc))
            wbuf[pl.ds(g * L, L)] = w
            gidx[pl.ds(g * L, L)] = rv * 2 + c
            gidx[pl.ds(CH + g * L, L)] = tv * 2 + (p3base + c)

    def _stage2(off, hb, ei):
        for g in range(CH // L):
            hv = icat[0, pl.ds(off + g * L, L)]
            hb[pl.ds(g * L, L)] = hv
            ei[pl.ds(g * L, L)] = lax.shift_right_logical(hv, 7)

    _stage1(0, 0, wbufA, gidxA)
    _stage2(0, hbA, eiA)
    pltpu.async_copy(ptab_hbm.at[gidxA], rowsA, semgA)
    pltpu.async_copy(rowsB.at[pl.ds(0, CH)], acc.at[hbB], semaB, add=True)
    pltpu.async_copy(rowsB.at[pl.ds(CH, CH)], acc_eb.at[eiB], sembB, add=True)

    def _body(k, off_next, bufX, bufY):
        (wbufX, hbX, eiX, gidxX, rowsX, semgX, semaX, sembX) = bufX
        (wbufY, hbY, eiY, gidxY, rowsY, semgY, semaY, sembY) = bufY
        rX_lo = rowsX.at[pl.ds(0, CH)]
        rX_hi = rowsX.at[pl.ds(CH, CH)]
        _stage1(k + 1, off_next, wbufY, gidxY)
        pltpu.make_async_copy(rowsY.at[pl.ds(0, CH)], acc.at[hbY],
                              semaY).wait()
        pltpu.make_async_copy(rowsY.at[pl.ds(CH, CH)], acc_eb.at[eiY],
                              sembY).wait()
        _stage2(off_next, hbY, eiY)
        pltpu.async_copy(ptab_hbm.at[gidxY], rowsY, semgY)
        pltpu.make_async_copy(ptab_hbm.at[gidxX], rowsX, semgX).wait()

        def _edge(e, _):
            wv = plsc.load_gather(wbufX, [jnp.full((L,), e, jnp.int32)])
            for j in range(DH // L):
                sl = pl.ds(j * L, L)
                rowsX[e, sl] = (rowsX[e, sl] + rowsX[CH + e, sl]) * wv
                rowsX[CH + e, sl] = zeros_f
            return 0
        lax.fori_loop(0, CH, _edge, 0, unroll=8)
        for g in range(CH // L):
            sl = pl.ds(g * L, L)
            colv = jnp.bitwise_and(hbX[sl], 127)
            plsc.store_scatter(rowsX, [CH + g * L + iota, colv], wbufX[sl])
        pltpu.async_copy(rX_lo, acc.at[hbX], semaX, add=True)
        pltpu.async_copy(rX_hi, acc_eb.at[eiX], sembX, add=True)

    bufA = (wbufA, hbA, eiA, gidxA, rowsA, semgA, semaA, sembA)
    bufB = (wbufB, hbB, eiB, gidxB, rowsB, semgB, semaB, sembB)

    def _pair(p, _):
        _body(2 * p, CH, bufA, bufB)
        _body(2 * p + 1, 0, bufB, bufA)
        return 0

    nch = EPT // CH
    lax.fori_loop(0, nch // 2, _pair, 0)
    pltpu.make_async_copy(ptab_hbm.at[gidxA], rowsA, semgA).wait()
    pltpu.make_async_copy(rowsB.at[pl.ds(0, CH)], acc.at[hbB], semaB).wait()
    pltpu.make_async_copy(rowsB.at[pl.ds(CH, CH)], acc_eb.at[eiB],
                          sembB).wait()
    plsc.subcore_barrier()
    def _copy_out(k, _):
        @pl.when(lax.rem(k, NS) == s)
        def _():
            rp = pl.ds(k * 2 * CH, 2 * CH)
            pltpu.sync_copy(acc.at[rp], out_hbm.at[c].at[rp])
        return 0
    lax.fori_loop(0, ACCR // (2 * CH), _copy_out, 0)

    @pl.when(s == 2)
    def _():
        rem = ACCR - (ACCR // (2 * CH)) * 2 * CH
        rp = pl.ds(ACCR - rem, rem)
        pltpu.sync_copy(acc.at[rp], out_hbm.at[c].at[rp])

    @pl.when(s == 0)
    def _():
        pltpu.sync_copy(acc_eb, eb_hbm.at[c])


def _sc_aggregate(idx3, q1, q2, q3, ptab):
    mesh = plsc.VectorSubcoreMesh(core_axis_name="c", subcore_axis_name="s",
                                  num_cores=NC, num_subcores=NS)
    fn = pl.kernel(
        _sc_body,
        out_type=[
            jax.ShapeDtypeStruct((NC, NPAD, DH), jnp.float32),
            jax.ShapeDtypeStruct((NC, EBR, DH), jnp.float32),
        ],
        mesh=mesh,
        scratch_types=[
            pltpu.VMEM((NQ,), jnp.float32),
            pltpu.VMEM((NQ,), jnp.float32),
            pltpu.VMEM((NQ,), jnp.float32),
            pltpu.VMEM((3, 2 * CH), jnp.int32),
            pltpu.VMEM((CH,), jnp.int32),
            pltpu.VMEM((CH,), jnp.int32),
            pltpu.VMEM((CH,), jnp.int32),
            pltpu.VMEM((CH,), jnp.int32),
            pltpu.VMEM((CH,), jnp.float32),
            pltpu.VMEM((CH,), jnp.float32),
            pltpu.VMEM((2 * CH,), jnp.int32),
            pltpu.VMEM((2 * CH,), jnp.int32),
            pltpu.VMEM((2 * CH, DH), jnp.float32),
            pltpu.VMEM((2 * CH, DH), jnp.float32),
            pltpu.VMEM_SHARED((ACCR, DH), jnp.float32),
            pltpu.VMEM_SHARED((EBR, DH), jnp.float32),
            pltpu.SemaphoreType.DMA,
            pltpu.SemaphoreType.DMA,
            pltpu.SemaphoreType.DMA,
            pltpu.SemaphoreType.DMA,
            pltpu.SemaphoreType.DMA,
            pltpu.SemaphoreType.DMA,
        ],
        compiler_params=pltpu.CompilerParams(needs_layout_passes=False),
    )
    return fn(idx3, q1, q2, q3, ptab)


def _epi_body(p1_ref, a0_ref, a1_ref, eb_ref, out_ref):
    am = jnp.concatenate([a0_ref[0], a1_ref[0]], axis=1)
    eb = eb_ref[...]
    hs = eb * p1_ref[...] + am
    ebs = jnp.where(eb == 0.0, jnp.float32(1e-12), eb)
    r = hs / ebs
    out_ref[...] = jnp.where(r > 0.0, r, jnp.exp(jnp.minimum(r, 0.0)) - 1.0)


def _epilogue(p1b, acc, ebcol):
    nblk = NPAD // BLK
    return pl.pallas_call(
        _epi_body,
        grid=(nblk,),
        in_specs=[
            pl.BlockSpec((BLK, OUT_DIM), lambda i: (i, 0)),
            pl.BlockSpec((1, BLK, DH), lambda i: (0, i, 0)),
            pl.BlockSpec((1, BLK, DH), lambda i: (1, i, 0)),
            pl.BlockSpec((BLK, 1), lambda i: (i, 0)),
        ],
        out_specs=pl.BlockSpec((BLK, OUT_DIM), lambda i: (i, 0)),
        out_shape=jax.ShapeDtypeStruct((NPAD, OUT_DIM), jnp.float32),
    )(p1b, acc, acc, ebcol)


def kernel(triplets, ent_embed, rel_embed, a_w, a_b, a2_w, a2_b):
    f32 = jnp.float32
    ent_p = jnp.zeros((NPAD, IN_DIM), f32).at[:N_ENT].set(ent_embed)
    rel_p = jnp.zeros((NPAD, IN_DIM), f32).at[:N_REL].set(rel_embed)
    w1 = a_w[:, :IN_DIM]
    w2 = a_w[:, IN_DIM:2 * IN_DIM]
    w3 = a_w[:, 2 * IN_DIM:]
    ab2d = a_b.reshape(1, OUT_DIM)
    a2row = a2_w.reshape(1, OUT_DIM)
    qb2d = a2_b.reshape(1, 1)

    p1b, p2, p3, q1t, q2t, q3t = _node_matmuls(ent_p, rel_p, w1, w2, w3,
                                               ab2d, a2row, qb2d)
    q1 = q1t[:NQ, 0]
    q2 = q2t[:NQ, 0]
    q3 = q3t[:NQ, 0]
    ptab = jnp.concatenate([p2.reshape(2 * NPAD, DH),
                            p3.reshape(2 * NPAD, DH)], axis=0)

    pad_h = jnp.full((EP - N_EDGES,), NQ - 1, jnp.int32)
    pad_z = jnp.zeros((EP - N_EDGES,), jnp.int32)
    heads = jnp.concatenate([triplets[:, 0], pad_h])
    tails = jnp.concatenate([triplets[:, 1], pad_z])
    rels = jnp.concatenate([triplets[:, 2], pad_z])
    npair = EPT // CH // 2
    idx3 = jnp.stack([heads.reshape(NS, npair, 2 * CH),
                      rels.reshape(NS, npair, 2 * CH),
                      tails.reshape(NS, npair, 2 * CH)], axis=2)
    idx3 = jnp.concatenate(
        [idx3, jnp.zeros((NS, 1, 3, 2 * CH), jnp.int32)], axis=1)

    acc, ebacc = _sc_aggregate(idx3, q1, q2, q3, ptab)
    ebcol = ebacc[0, :NPAD // DH].reshape(NPAD, 1)
    out = _epilogue(p1b, acc, ebcol)
    return out[:N_ENT]


# speedup vs baseline: 1.3795x; 1.3795x over previous
import jax
import jax.numpy as jnp
from jax import lax
from jax.experimental import pallas as pl
from jax.experimental.pallas import tpu as pltpu
from jax.experimental.pallas import tpu_sc as plsc

N_ENT = 10000
N_REL = 10000
IN_DIM = 256
OUT_DIM = 256
N_EDGES = 160000

NPAD = 10240
NC, NS, L = 2, 16, 16
CH = 32
EPT = 10240
EP = EPT * NS
DH = 128
BLK = 512
EBR = NPAD // DH
NQ = 10016
ACCR = 10016


def _mm_body(ent_ref, rel_ref, w1_ref, w2_ref, w3_ref, ab_ref, a2_ref,
             qb_ref, p1_ref, p2_ref, p3_ref, q1_ref, q2_ref, q3_ref):
    ent = ent_ref[...]
    rel = rel_ref[...]
    dn = (((1,), (1,)), ((), ()))
    p1 = lax.dot_general(ent, w1_ref[...], dn,
                         preferred_element_type=jnp.float32) + ab_ref[...]
    p2 = lax.dot_general(rel, w2_ref[...], dn,
                         preferred_element_type=jnp.float32)
    p3 = lax.dot_general(ent, w3_ref[...], dn,
                         preferred_element_type=jnp.float32)
    p1_ref[...] = p1
    p2_ref[...] = p2
    p3_ref[...] = p3
    a2 = a2_ref[...]
    q1 = jnp.sum(p1 * a2, axis=1, keepdims=True) + qb_ref[...]
    q2 = jnp.sum(p2 * a2, axis=1, keepdims=True)
    q3 = jnp.sum(p3 * a2, axis=1, keepdims=True)
    q1_ref[...] = jnp.broadcast_to(q1, (BLK, L))
    q2_ref[...] = jnp.broadcast_to(q2, (BLK, L))
    q3_ref[...] = jnp.broadcast_to(q3, (BLK, L))


def _node_matmuls(ent_p, rel_p, w1, w2, w3, ab2d, a2row, qb2d):
    nblk = NPAD // BLK
    return pl.pallas_call(
        _mm_body,
        grid=(nblk,),
        in_specs=[
            pl.BlockSpec((BLK, IN_DIM), lambda i: (i, 0)),
            pl.BlockSpec((BLK, IN_DIM), lambda i: (i, 0)),
            pl.BlockSpec((OUT_DIM, IN_DIM), lambda i: (0, 0)),
            pl.BlockSpec((OUT_DIM, IN_DIM), lambda i: (0, 0)),
            pl.BlockSpec((OUT_DIM, IN_DIM), lambda i: (0, 0)),
            pl.BlockSpec((1, OUT_DIM), lambda i: (0, 0)),
            pl.BlockSpec((1, OUT_DIM), lambda i: (0, 0)),
            pl.BlockSpec((1, 1), lambda i: (0, 0)),
        ],
        out_specs=[
            pl.BlockSpec((BLK, OUT_DIM), lambda i: (i, 0)),
            pl.BlockSpec((BLK, OUT_DIM), lambda i: (i, 0)),
            pl.BlockSpec((BLK, OUT_DIM), lambda i: (i, 0)),
            pl.BlockSpec((BLK, L), lambda i: (i, 0)),
            pl.BlockSpec((BLK, L), lambda i: (i, 0)),
            pl.BlockSpec((BLK, L), lambda i: (i, 0)),
        ],
        out_shape=[
            jax.ShapeDtypeStruct((NPAD, OUT_DIM), jnp.float32),
            jax.ShapeDtypeStruct((NPAD, OUT_DIM), jnp.float32),
            jax.ShapeDtypeStruct((NPAD, OUT_DIM), jnp.float32),
            jax.ShapeDtypeStruct((NPAD, L), jnp.float32),
            jax.ShapeDtypeStruct((NPAD, L), jnp.float32),
            jax.ShapeDtypeStruct((NPAD, L), jnp.float32),
        ],
    )(ent_p, rel_p, w1, w2, w3, ab2d, a2row, qb2d)


def _sc_body(idx3_hbm, q1_hbm, q2_hbm, q3_hbm, ptab_hbm, out_hbm, eb_hbm,
             q1v, q2v, q3v, icat, hbA, hbB, eiA, eiB, wbufA, wbufB,
             gidxA, gidxB, rowsA, rowsB, acc, acc_eb,
             semgA, semgB, semaA, semaB, sembA, sembB):
    c = lax.axis_index("c")
    s = lax.axis_index("s")
    iota = lax.iota(jnp.int32, L)
    zeros_i = jnp.zeros((L,), jnp.int32)
    zeros_f = jnp.zeros((L,), jnp.float32)
    p3base = 2 * NPAD

    pltpu.sync_copy(q1_hbm, q1v)
    pltpu.sync_copy(q2_hbm, q2v)
    pltpu.sync_copy(q3_hbm, q3v)

    def _zero_row(e, _):
        for j in range(DH // L):
            rowsA[e, pl.ds(j * L, L)] = zeros_f
            rowsB[e, pl.ds(j * L, L)] = zeros_f
        return 0
    lax.fori_loop(0, 2 * CH, _zero_row, 0)

    def _zero_acc(k, _):
        @pl.when(lax.rem(k, NS) == s)
        def _():
            pltpu.sync_copy(rowsA, acc.at[pl.ds(k * 2 * CH, 2 * CH)])
        return 0
    lax.fori_loop(0, ACCR // (2 * CH), _zero_acc, 0)

    @pl.when(s == 1)
    def _():
        rem = ACCR - (ACCR // (2 * CH)) * 2 * CH
        pltpu.sync_copy(rowsA.at[pl.ds(0, rem)],
                        acc.at[pl.ds(ACCR - rem, rem)])

    @pl.when(s == 0)
    def _():
        pltpu.sync_copy(rowsA, acc_eb.at[pl.ds(0, 2 * CH)])
        pltpu.sync_copy(rowsA.at[pl.ds(0, EBR - 2 * CH)],
                        acc_eb.at[pl.ds(2 * CH, EBR - 2 * CH)])
    for g in range(CH // L):
        sl = pl.ds(g * L, L)
        hbB[sl] = zeros_i
        eiB[sl] = zeros_i
    plsc.subcore_barrier()

    def _stage1(k, off, wbuf, gidx):
        if off == 0:
            pltpu.sync_copy(idx3_hbm.at[s].at[k // 2], icat)
        for g in range(CH // L):
            sl = pl.ds(off + g * L, L)
            hv = icat[0, sl]
            rv = icat[1, sl]
            tv = icat[2, sl]
            sc = (plsc.load_gather(q1v, [hv]) +
                  plsc.load_gather(q2v, [rv]) +
                  plsc.load_gather(q3v, [tv]))
            w = jnp.exp(jnp.maximum(sc, 0.01 * sc))
            wbuf[pl.ds(g * L, L)] = w
            gidx[pl.ds(g * L, L)] = rv * 2 + c
            gidx[pl.ds(CH + g * L, L)] = tv * 2 + (p3base + c)

    def _stage2(off, hb, ei):
        for g in range(CH // L):
            hv = icat[0, pl.ds(off + g * L, L)]
            hb[pl.ds(g * L, L)] = hv
            ei[pl.ds(g * L, L)] = lax.shift_right_logical(hv, 7)

    _stage1(0, 0, wbufA, gidxA)
    _stage2(0, hbA, eiA)
    pltpu.async_copy(ptab_hbm.at[gidxA], rowsA, semgA)
    pltpu.async_copy(rowsB.at[pl.ds(0, CH)], acc.at[hbB], semaB, add=True)
    pltpu.async_copy(rowsB.at[pl.ds(CH, CH)], acc_eb.at[eiB], sembB, add=True)

    def _body(k, off_next, bufX, bufY):
        (wbufX, hbX, eiX, gidxX, rowsX, semgX, semaX, sembX) = bufX
        (wbufY, hbY, eiY, gidxY, rowsY, semgY, semaY, sembY) = bufY
        rX_lo = rowsX.at[pl.ds(0, CH)]
        rX_hi = rowsX.at[pl.ds(CH, CH)]
        _stage1(k + 1, off_next, wbufY, gidxY)
        pltpu.make_async_copy(rowsY.at[pl.ds(0, CH)], acc.at[hbY],
                              semaY).wait()
        pltpu.make_async_copy(rowsY.at[pl.ds(CH, CH)], acc_eb.at[eiY],
                              sembY).wait()
        _stage2(off_next, hbY, eiY)
        pltpu.async_copy(ptab_hbm.at[gidxY], rowsY, semgY)
        pltpu.make_async_copy(ptab_hbm.at[gidxX], rowsX, semgX).wait()

        def _edge(e, _):
            wv = plsc.load_gather(wbufX, [jnp.full((L,), e, jnp.int32)])
            for j in range(DH // L):
                sl = pl.ds(j * L, L)
                rowsX[e, sl] = (rowsX[e, sl] + rowsX[CH + e, sl]) * wv
                rowsX[CH + e, sl] = zeros_f
            return 0
        lax.fori_loop(0, CH, _edge, 0, unroll=4)
        for g in range(CH // L):
            sl = pl.ds(g * L, L)
            colv = jnp.bitwise_and(hbX[sl], 127)
            plsc.store_scatter(rowsX, [CH + g * L + iota, colv], wbufX[sl])
        pltpu.async_copy(rX_lo, acc.at[hbX], semaX, add=True)
        pltpu.async_copy(rX_hi, acc_eb.at[eiX], sembX, add=True)

    bufA = (wbufA, hbA, eiA, gidxA, rowsA, semgA, semaA, sembA)
    bufB = (wbufB, hbB, eiB, gidxB, rowsB, semgB, semaB, sembB)

    def _pair(p, _):
        _body(2 * p, CH, bufA, bufB)
        _body(2 * p + 1, 0, bufB, bufA)
        return 0

    nch = EPT // CH
    lax.fori_loop(0, nch // 2, _pair, 0)
    pltpu.make_async_copy(ptab_hbm.at[gidxA], rowsA, semgA).wait()
    pltpu.make_async_copy(rowsB.at[pl.ds(0, CH)], acc.at[hbB], semaB).wait()
    pltpu.make_async_copy(rowsB.at[pl.ds(CH, CH)], acc_eb.at[eiB],
                          sembB).wait()
    plsc.subcore_barrier()
    def _copy_out(k, _):
        @pl.when(lax.rem(k, NS) == s)
        def _():
            rp = pl.ds(k * 2 * CH, 2 * CH)
            pltpu.sync_copy(acc.at[rp], out_hbm.at[c].at[rp])
        return 0
    lax.fori_loop(0, ACCR // (2 * CH), _copy_out, 0)

    @pl.when(s == 2)
    def _():
        rem = ACCR - (ACCR // (2 * CH)) * 2 * CH
        rp = pl.ds(ACCR - rem, rem)
        pltpu.sync_copy(acc.at[rp], out_hbm.at[c].at[rp])

    @pl.when(s == 0)
    def _():
        pltpu.sync_copy(acc_eb, eb_hbm.at[c])


def _sc_aggregate(idx3, q1, q2, q3, ptab):
    mesh = plsc.VectorSubcoreMesh(core_axis_name="c", subcore_axis_name="s",
                                  num_cores=NC, num_subcores=NS)
    fn = pl.kernel(
        _sc_body,
        out_type=[
            jax.ShapeDtypeStruct((NC, NPAD, DH), jnp.float32),
            jax.ShapeDtypeStruct((NC, EBR, DH), jnp.float32),
        ],
        mesh=mesh,
        scratch_types=[
            pltpu.VMEM((NQ,), jnp.float32),
            pltpu.VMEM((NQ,), jnp.float32),
            pltpu.VMEM((NQ,), jnp.float32),
            pltpu.VMEM((3, 2 * CH), jnp.int32),
            pltpu.VMEM((CH,), jnp.int32),
            pltpu.VMEM((CH,), jnp.int32),
            pltpu.VMEM((CH,), jnp.int32),
            pltpu.VMEM((CH,), jnp.int32),
            pltpu.VMEM((CH,), jnp.float32),
            pltpu.VMEM((CH,), jnp.float32),
            pltpu.VMEM((2 * CH,), jnp.int32),
            pltpu.VMEM((2 * CH,), jnp.int32),
            pltpu.VMEM((2 * CH, DH), jnp.float32),
            pltpu.VMEM((2 * CH, DH), jnp.float32),
            pltpu.VMEM_SHARED((ACCR, DH), jnp.float32),
            pltpu.VMEM_SHARED((EBR, DH), jnp.float32),
            pltpu.SemaphoreType.DMA,
            pltpu.SemaphoreType.DMA,
            pltpu.SemaphoreType.DMA,
            pltpu.SemaphoreType.DMA,
            pltpu.SemaphoreType.DMA,
            pltpu.SemaphoreType.DMA,
        ],
        compiler_params=pltpu.CompilerParams(needs_layout_passes=False),
    )
    return fn(idx3, q1, q2, q3, ptab)


def _epi_body(p1_ref, a0_ref, a1_ref, eb_ref, out_ref):
    am = jnp.concatenate([a0_ref[0], a1_ref[0]], axis=1)
    eb = eb_ref[...]
    hs = eb * p1_ref[...] + am
    ebs = jnp.where(eb == 0.0, jnp.float32(1e-12), eb)
    r = hs / ebs
    out_ref[...] = jnp.where(r > 0.0, r, jnp.exp(jnp.minimum(r, 0.0)) - 1.0)


def _epilogue(p1b, acc, ebcol):
    nblk = NPAD // BLK
    return pl.pallas_call(
        _epi_body,
        grid=(nblk,),
        in_specs=[
            pl.BlockSpec((BLK, OUT_DIM), lambda i: (i, 0)),
            pl.BlockSpec((1, BLK, DH), lambda i: (0, i, 0)),
            pl.BlockSpec((1, BLK, DH), lambda i: (1, i, 0)),
            pl.BlockSpec((BLK, 1), lambda i: (i, 0)),
        ],
        out_specs=pl.BlockSpec((BLK, OUT_DIM), lambda i: (i, 0)),
        out_shape=jax.ShapeDtypeStruct((NPAD, OUT_DIM), jnp.float32),
    )(p1b, acc, acc, ebcol)


def kernel(triplets, ent_embed, rel_embed, a_w, a_b, a2_w, a2_b):
    f32 = jnp.float32
    ent_p = jnp.zeros((NPAD, IN_DIM), f32).at[:N_ENT].set(ent_embed)
    rel_p = jnp.zeros((NPAD, IN_DIM), f32).at[:N_REL].set(rel_embed)
    w1 = a_w[:, :IN_DIM]
    w2 = a_w[:, IN_DIM:2 * IN_DIM]
    w3 = a_w[:, 2 * IN_DIM:]
    ab2d = a_b.reshape(1, OUT_DIM)
    a2row = a2_w.reshape(1, OUT_DIM)
    qb2d = a2_b.reshape(1, 1)

    p1b, p2, p3, q1t, q2t, q3t = _node_matmuls(ent_p, rel_p, w1, w2, w3,
                                               ab2d, a2row, qb2d)
    q1 = q1t[:NQ, 0]
    q2 = q2t[:NQ, 0]
    q3 = q3t[:NQ, 0]
    ptab = jnp.concatenate([p2.reshape(2 * NPAD, DH),
                            p3.reshape(2 * NPAD, DH)], axis=0)

    pad_h = jnp.full((EP - N_EDGES,), NQ - 1, jnp.int32)
    pad_z = jnp.zeros((EP - N_EDGES,), jnp.int32)
    heads = jnp.concatenate([triplets[:, 0], pad_h])
    tails = jnp.concatenate([triplets[:, 1], pad_z])
    rels = jnp.concatenate([triplets[:, 2], pad_z])
    npair = EPT // CH // 2
    idx3 = jnp.stack([heads.reshape(NS, npair, 2 * CH),
                      rels.reshape(NS, npair, 2 * CH),
                      tails.reshape(NS, npair, 2 * CH)], axis=2)
    idx3 = jnp.concatenate(
        [idx3, jnp.zeros((NS, 1, 3, 2 * CH), jnp.int32)], axis=1)

    acc, ebacc = _sc_aggregate(idx3, q1, q2, q3, ptab)
    ebcol = ebacc[0, :NPAD // DH].reshape(NPAD, 1)
    out = _epilogue(p1b, acc, ebcol)
    return out[:N_ENT]
